# Initial kernel scaffold; baseline (speedup 1.0000x reference)
import jax
import jax.numpy as jnp
from jax.experimental import pallas as pl


def kernel(feat, rgt_map, W1, b1, W2, b2, gamma, beta, gate):
    raise NotImplementedError("write your pallas kernel here")


# speedup vs baseline: 27.8758x; 27.8758x over previous
import jax
import jax.numpy as jnp
from jax.experimental import pallas as pl

SIGMA = 0.08
H = 224
W = 224
N = H * W
C = 128
R = 3584
NT = N // R
HALO = 900
P = R + 2 * HALO


def _maps_kernel(r_ref, wu_ref, wd_ref, wl_ref, wr_ref, s_ref, s2_ref):
    r = r_ref[...]
    rmin = jnp.min(r)
    rmax = jnp.max(r)
    r01 = (r - rmin) / jnp.maximum(rmax - rmin, 1e-12)
    inv = 1.0 / (2.0 * SIGMA * SIGMA + 1e-12)
    dv = r01[1:, :] - r01[:-1, :]
    wv = jnp.exp(-(dv * dv) * inv)
    zrow = jnp.zeros((1, W), jnp.float32)
    wdn = jnp.concatenate([wv, zrow], axis=0)
    wup = jnp.concatenate([zrow, wv], axis=0)
    dh = r01[:, 1:] - r01[:, :-1]
    wh = jnp.exp(-(dh * dh) * inv)
    zcol = jnp.zeros((H, 1), jnp.float32)
    wrt = jnp.concatenate([wh, zcol], axis=1)
    wlf = jnp.concatenate([zcol, wh], axis=1)
    deg = wup + wdn + wlf + wrt
    wu_ref[...] = wup
    wd_ref[...] = wdn
    wl_ref[...] = wlf
    wr_ref[...] = wrt
    s_ref[...] = (deg + 1e-12) ** -0.5
    s2_ref[...] = (deg + 1.0) ** -0.5


def _sh(a, k):
    z = jnp.zeros((abs(k), a.shape[1]), a.dtype)
    if k > 0:
        return jnp.concatenate([a[k:], z], axis=0)
    return jnp.concatenate([z, a[:k]], axis=0)


def _stencil(q, wu, wd, wl, wr, s):
    return s * (wu * _sh(q, -W) + wd * _sh(q, W)
                + wl * _sh(q, -1) + wr * _sh(q, 1))


def _main_kernel(xp_ref, xc_ref, xn_ref, mp_ref, mc_ref, mn_ref,
                 w1_ref, b1_ref, w2_ref, b2_ref, gam_ref, bet_ref, gate_ref,
                 o_ref):
    x3 = jnp.concatenate([xp_ref[...], xc_ref[...], xn_ref[...]], axis=0)
    m3 = jnp.concatenate([mp_ref[...], mc_ref[...], mn_ref[...]], axis=0)
    xpad = x3[R - HALO:2 * R + HALO]
    m = m3[R - HALO:2 * R + HALO]
    wu = m[:, 0:1]
    wd = m[:, 1:2]
    wl = m[:, 2:3]
    wr = m[:, 3:4]
    s = m[:, 4:5]
    s2 = m[:, 5:6]

    d1 = _stencil(s * xpad, wu, wd, wl, wr, s)
    d2 = _stencil(s * d1, wu, wd, wl, wr, s)

    a, b = 450, P - 450
    w1 = w1_ref[...]
    dot = lambda u, v: jnp.dot(u, v, preferred_element_type=jnp.float32,
                               precision=jax.lax.Precision.HIGHEST)
    z1 = (dot(xpad[a:b], w1[0:C])
          + dot(d1[a:b], w1[C:2 * C])
          + dot(d2[a:b], w1[2 * C:3 * C]))

    wu1, wd1, wl1, wr1 = wu[a:b], wd[a:b], wl[a:b], wr[a:b]
    s21 = s2[a:b]
    y1 = (_stencil(s21 * z1, wu1, wd1, wl1, wr1, s21)
          + (s21 * s21) * z1 + b1_ref[...])

    mu = jnp.mean(y1, axis=-1, keepdims=True)
    var = jnp.mean((y1 - mu) ** 2, axis=-1, keepdims=True)
    y = (y1 - mu) * jax.lax.rsqrt(var + 1e-5) * gam_ref[...] + bet_ref[...]
    y = jnp.maximum(y, 0.0)

    z2 = dot(y[225:-225], w2_ref[...])
    a2, b2i = 675, P - 675
    wu2, wd2, wl2, wr2 = wu[a2:b2i], wd[a2:b2i], wl[a2:b2i], wr[a2:b2i]
    s22 = s2[a2:b2i]
    y2 = (_stencil(s22 * z2, wu2, wd2, wl2, wr2, s22)
          + (s22 * s22) * z2 + b2_ref[...])
    y2c = y2[225:-225]

    xc = xc_ref[...]
    g = jnp.clip(gate_ref[0, 0], 0.0, 1.0)
    o_ref[...] = xc + g * (y2c - xc)


def kernel(feat, rgt_map, W1, b1, W2, b2, gamma, beta, gate):
    r2d = rgt_map.reshape(H, W)
    wu, wd, wl, wr, s, s2 = pl.pallas_call(
        _maps_kernel,
        out_shape=[jax.ShapeDtypeStruct((H, W), jnp.float32)] * 6,
    )(r2d)
    maps = jnp.stack([wu, wd, wl, wr, s, s2], axis=-1).reshape(N, 6)

    x2d = jnp.transpose(feat, (0, 2, 3, 1)).reshape(N, C)

    xspec = lambda f: pl.BlockSpec((R, C), lambda i: (f(i), 0))
    mspec = lambda f: pl.BlockSpec((R, 6), lambda i: (f(i), 0))
    prev = lambda i: jnp.maximum(i - 1, 0)
    nxt = lambda i: jnp.minimum(i + 1, NT - 1)
    cur = lambda i: i
    full = lambda r, c: pl.BlockSpec((r, c), lambda i: (0, 0))

    out_rows = pl.pallas_call(
        _main_kernel,
        grid=(NT,),
        in_specs=[
            xspec(prev), xspec(cur), xspec(nxt),
            mspec(prev), mspec(cur), mspec(nxt),
            full(3 * C, C), full(1, C), full(C, C), full(1, C),
            full(1, C), full(1, C), full(1, 1),
        ],
        out_specs=pl.BlockSpec((R, C), lambda i: (i, 0)),
        out_shape=jax.ShapeDtypeStruct((N, C), jnp.float32),
    )(x2d, x2d, x2d, maps, maps, maps,
      W1, b1.reshape(1, C), W2, b2.reshape(1, C),
      gamma.reshape(1, C), beta.reshape(1, C),
      jnp.asarray(gate, jnp.float32).reshape(1, 1))

    return jnp.transpose(out_rows.reshape(1, H, W, C), (0, 3, 1, 2))


# speedup vs baseline: 63.3352x; 2.2720x over previous
import jax
import jax.numpy as jnp
from jax.experimental import pallas as pl

SIGMA = 0.08
H = 224
W = 224
N = H * W
C = 128
R = 3584
NT = N // R
HALO = 904
P = R + 2 * HALO
NM = 9


def _sh2(a, dx, dy):
    if dx == 1:
        a = jnp.concatenate([a[1:, :], jnp.zeros((1, W), a.dtype)], axis=0)
    elif dx == -1:
        a = jnp.concatenate([jnp.zeros((1, W), a.dtype), a[:-1, :]], axis=0)
    if dy == 1:
        a = jnp.concatenate([a[:, 1:], jnp.zeros((H, 1), a.dtype)], axis=1)
    elif dy == -1:
        a = jnp.concatenate([jnp.zeros((H, 1), a.dtype), a[:, :-1]], axis=1)
    return a


def _maps_kernel(r_ref, o_ref):
    r = r_ref[...]
    rmin = jnp.min(r)
    rmax = jnp.max(r)
    r01 = (r - rmin) / jnp.maximum(rmax - rmin, 1e-12)
    inv = 1.0 / (2.0 * SIGMA * SIGMA + 1e-12)
    dv = r01[1:, :] - r01[:-1, :]
    wv = jnp.exp(-(dv * dv) * inv)
    zrow = jnp.zeros((1, W), jnp.float32)
    wdn = jnp.concatenate([wv, zrow], axis=0)
    wup = jnp.concatenate([zrow, wv], axis=0)
    dh = r01[:, 1:] - r01[:, :-1]
    wh = jnp.exp(-(dh * dh) * inv)
    zcol = jnp.zeros((H, 1), jnp.float32)
    wrt = jnp.concatenate([wh, zcol], axis=1)
    wlf = jnp.concatenate([zcol, wh], axis=1)
    deg = wup + wdn + wlf + wrt
    s = (deg + 1e-12) ** -0.5
    s2 = (deg + 1.0) ** -0.5
    o_ref[0] = wup * s * _sh2(s, -1, 0)
    o_ref[1] = wdn * s * _sh2(s, 1, 0)
    o_ref[2] = wlf * s * _sh2(s, 0, -1)
    o_ref[3] = wrt * s * _sh2(s, 0, 1)
    o_ref[4] = wup * s2 * _sh2(s2, -1, 0)
    o_ref[5] = wdn * s2 * _sh2(s2, 1, 0)
    o_ref[6] = wlf * s2 * _sh2(s2, 0, -1)
    o_ref[7] = wrt * s2 * _sh2(s2, 0, 1)
    o_ref[8] = s2 * s2


def _sh(a, k):
    z = jnp.zeros((abs(k), a.shape[1]), a.dtype)
    if k > 0:
        return jnp.concatenate([a[k:], z], axis=0)
    return jnp.concatenate([z, a[:k]], axis=0)


def _agg(q, wu, wd, wl, wr):
    return (wu * _sh(q, -W) + wd * _sh(q, W)
            + wl * _sh(q, -1) + wr * _sh(q, 1))


def _main_kernel(xp_ref, xc_ref, xn_ref, mp_ref, mc_ref, mn_ref,
                 w1_ref, b1_ref, w2_ref, b2_ref, gam_ref, bet_ref, gate_ref,
                 o_ref):
    x3 = jnp.concatenate([xp_ref[...], xc_ref[...], xn_ref[...]], axis=0)
    m3 = jnp.concatenate([mp_ref[...], mc_ref[...], mn_ref[...]], axis=0)
    xpad = x3[R - HALO:2 * R + HALO]
    m = m3[R - HALO:2 * R + HALO]
    du = m[:, 0:1]
    dd = m[:, 1:2]
    dl = m[:, 2:3]
    dr = m[:, 3:4]

    d1 = _agg(xpad, du, dd, dl, dr)
    d2 = _agg(d1, du, dd, dl, dr)

    a1 = 448
    b1i = P - 448
    w1 = w1_ref[...]
    dot = lambda u, v: jnp.dot(u, v, preferred_element_type=jnp.float32)
    z1 = (dot(xpad[a1:b1i], w1[0:C])
          + dot(d1[a1:b1i], w1[C:2 * C])
          + dot(d2[a1:b1i], w1[2 * C:3 * C]))

    gu, gd = m[a1:b1i, 4:5], m[a1:b1i, 5:6]
    gl, gr = m[a1:b1i, 6:7], m[a1:b1i, 7:8]
    gs = m[a1:b1i, 8:9]
    y1 = _agg(z1, gu, gd, gl, gr) + gs * z1 + b1_ref[...]

    mu = jnp.mean(y1, axis=-1, keepdims=True)
    var = jnp.mean((y1 - mu) ** 2, axis=-1, keepdims=True)
    y = (y1 - mu) * jax.lax.rsqrt(var + 1e-5) * gam_ref[...] + bet_ref[...]
    y = jnp.maximum(y, 0.0)

    z2 = dot(y[224:-224], w2_ref[...])
    a2 = 672
    b2i = P - 672
    hu, hd = m[a2:b2i, 4:5], m[a2:b2i, 5:6]
    hl, hr = m[a2:b2i, 6:7], m[a2:b2i, 7:8]
    hs = m[a2:b2i, 8:9]
    y2 = _agg(z2, hu, hd, hl, hr) + hs * z2 + b2_ref[...]
    y2c = y2[232:-232]

    xc = xc_ref[...]
    g = jnp.clip(gate_ref[0, 0], 0.0, 1.0)
    o_ref[...] = xc + g * (y2c - xc)


def kernel(feat, rgt_map, W1, b1, W2, b2, gamma, beta, gate):
    r2d = rgt_map.reshape(H, W)
    maps3 = pl.pallas_call(
        _maps_kernel,
        out_shape=jax.ShapeDtypeStruct((NM, H, W), jnp.float32),
    )(r2d)
    maps = jnp.transpose(maps3.reshape(NM, N), (1, 0))

    x2d = jnp.transpose(feat, (0, 2, 3, 1)).reshape(N, C)

    xspec = lambda f: pl.BlockSpec((R, C), lambda i, f=f: (f(i), 0))
    mspec = lambda f: pl.BlockSpec((R, NM), lambda i, f=f: (f(i), 0))
    prev = lambda i: jnp.maximum(i - 1, 0)
    nxt = lambda i: jnp.minimum(i + 1, NT - 1)
    cur = lambda i: i
    full = lambda r, c: pl.BlockSpec((r, c), lambda i: (0, 0))

    out_rows = pl.pallas_call(
        _main_kernel,
        grid=(NT,),
        in_specs=[
            xspec(prev), xspec(cur), xspec(nxt),
            mspec(prev), mspec(cur), mspec(nxt),
            full(3 * C, C), full(1, C), full(C, C), full(1, C),
            full(1, C), full(1, C), full(1, 1),
        ],
        out_specs=pl.BlockSpec((R, C), lambda i: (i, 0)),
        out_shape=jax.ShapeDtypeStruct((N, C), jnp.float32),
    )(x2d, x2d, x2d, maps, maps, maps,
      W1, b1.reshape(1, C), W2, b2.reshape(1, C),
      gamma.reshape(1, C), beta.reshape(1, C),
      jnp.asarray(gate, jnp.float32).reshape(1, 1))

    return jnp.transpose(out_rows.reshape(1, H, W, C), (0, 3, 1, 2))
